# Initial kernel scaffold; baseline (speedup 1.0000x reference)
import jax
import jax.numpy as jnp
from jax.experimental import pallas as pl


def kernel(x, edge_index, edge_weight, W1_rel, b1_rel, W1_root, W2_rel, b2_rel, W2_root):
    raise NotImplementedError("write your pallas kernel here")


# speedup vs baseline: 2.7135x; 2.7135x over previous
import functools

import jax
import jax.numpy as jnp
from jax import lax
from jax.experimental import pallas as pl
from jax.experimental.pallas import tpu as pltpu
from jax.experimental.pallas import tpu_sc as plsc

_NC = 2
_NS = 16
_NW = _NC * _NS
_LANES = 16
_CHUNK = 128


def _sc_segment_sum(x, src2d, dst2d, ew2d, n_pad, chunks_per_tile):
    d = x.shape[1]
    rows_per_tile = n_pad // _NS
    mesh = plsc.VectorSubcoreMesh(core_axis_name="c", subcore_axis_name="s")

    @functools.partial(
        pl.kernel,
        out_type=jax.ShapeDtypeStruct((_NC, n_pad, d), jnp.float32),
        mesh=mesh,
        scratch_types=[
            pltpu.VMEM((chunks_per_tile, _CHUNK), jnp.int32),
            pltpu.VMEM((chunks_per_tile, _CHUNK), jnp.int32),
            pltpu.VMEM((chunks_per_tile, _CHUNK), jnp.float32),
            pltpu.VMEM((_CHUNK, 128), jnp.float32),
            pltpu.VMEM_SHARED((n_pad, 128), jnp.float32),
        ],
    )
    def k(x_hbm, src_hbm, dst_hbm, ew_hbm, out_hbm,
          src_v, dst_v, ew_v, rows, accum):
        c = lax.axis_index("c")
        s = lax.axis_index("s")
        t = c * _NS + s

        base = t * chunks_per_tile
        pltpu.sync_copy(src_hbm.at[pl.ds(base, chunks_per_tile)], src_v)
        pltpu.sync_copy(dst_hbm.at[pl.ds(base, chunks_per_tile)], dst_v)
        pltpu.sync_copy(ew_hbm.at[pl.ds(base, chunks_per_tile)], ew_v)

        @pl.loop(0, _CHUNK)
        def _(r):
            for g in range(d // _LANES):
                rows[r, pl.ds(g * _LANES, _LANES)] = jnp.zeros((_LANES,), jnp.float32)

        rbase = s * rows_per_tile

        @pl.loop(0, rows_per_tile // _CHUNK)
        def _(i):
            pltpu.sync_copy(rows, accum.at[pl.ds(rbase + i * _CHUNK, _CHUNK)])

        rem = rows_per_tile % _CHUNK
        if rem:
            pltpu.sync_copy(
                rows.at[pl.ds(0, rem)],
                accum.at[pl.ds(rbase + (rows_per_tile // _CHUNK) * _CHUNK, rem)])

        plsc.subcore_barrier()

        @pl.loop(0, chunks_per_tile)
        def _(j):
            pltpu.sync_copy(x_hbm.at[src_v.at[j]], rows)

            @pl.loop(0, _CHUNK // _LANES)
            def _(g):
                raw = ew_v[j, pl.ds(g * _LANES, _LANES)]
                ew16 = 1.0 / (1.0 + jnp.exp(-raw))
                for le in range(_LANES):
                    e = g * _LANES + le
                    w = lax.gather(
                        ew16, jnp.full((_LANES, 1), le, jnp.int32),
                        lax.GatherDimensionNumbers(
                            offset_dims=(), collapsed_slice_dims=(0,),
                            start_index_map=(0,)),
                        (1,), mode=lax.GatherScatterMode.PROMISE_IN_BOUNDS)
                    for kk in range(d // _LANES):
                        sl = pl.ds(kk * _LANES, _LANES)
                        rows[e, sl] = rows[e, sl] * w

            pltpu.sync_copy(rows, accum.at[dst_v.at[j]], add=True)

        plsc.subcore_barrier()

        pltpu.sync_copy(accum.at[pl.ds(rbase, rows_per_tile)],
                        out_hbm.at[c].at[pl.ds(rbase, rows_per_tile)])

    return k(x, src2d, dst2d, ew2d)


def _tc_layer(aggp, inp, w_rel, b_rel, w_root, relu):
    n, d = inp.shape
    o = w_rel.shape[1]

    def body(aggp_ref, x_ref, wrel_ref, b_ref, wroot_ref, o_ref):
        agg = aggp_ref[0] + aggp_ref[1]
        r = (jnp.dot(agg, wrel_ref[...], preferred_element_type=jnp.float32,
                     precision=lax.Precision.HIGHEST)
             + jnp.dot(x_ref[...], wroot_ref[...],
                       preferred_element_type=jnp.float32,
                       precision=lax.Precision.HIGHEST)
             + b_ref[...])
        o_ref[...] = jnp.maximum(r, 0.0) if relu else r

    return pl.pallas_call(
        body,
        out_shape=jax.ShapeDtypeStruct((n, o), jnp.float32),
    )(aggp, inp, w_rel, b_rel.reshape(1, o), w_root)


def kernel(x, edge_index, edge_weight, W1_rel, b1_rel, W1_root,
           W2_rel, b2_rel, W2_root):
    n, d = x.shape
    e = edge_weight.shape[0]
    cpt = -(-e // (_NW * _CHUNK))
    cpt = -(-cpt // 8) * 8
    e_pad = _NW * cpt * _CHUNK

    src = jnp.pad(edge_index[0], (0, e_pad - e)).reshape(_NW * cpt, _CHUNK)
    dst = jnp.pad(edge_index[1], (0, e_pad - e)).reshape(_NW * cpt, _CHUNK)
    ewp = jnp.pad(edge_weight, (0, e_pad - e),
                  constant_values=-1e30).reshape(_NW * cpt, _CHUNK)

    n_pad = -(-n // (_NS * 8)) * (_NS * 8)

    p1 = _sc_segment_sum(x, src, dst, ewp, n_pad, cpt)[:, :n, :]
    h = _tc_layer(p1, x, W1_rel, b1_rel, W1_root, relu=True)
    p2 = _sc_segment_sum(h, src, dst, ewp, n_pad, cpt)[:, :n, :]
    out = _tc_layer(p2, h, W2_rel, b2_rel, W2_root, relu=False)
    return out


# speedup vs baseline: 3.4963x; 1.2885x over previous
import functools

import jax
import jax.numpy as jnp
from jax import lax
from jax.experimental import pallas as pl
from jax.experimental.pallas import tpu as pltpu
from jax.experimental.pallas import tpu_sc as plsc

_NC = 2
_NS = 16
_NW = _NC * _NS
_LANES = 16
_CHUNK = 128


_GRP = 8


def _sc_segment_sum(x, src4d, dst4d, ew4d, n_pad, chunks_per_tile):
    d = x.shape[1]
    rows_per_tile = n_pad // _NS
    n_grp = chunks_per_tile // _GRP
    assert n_grp % 2 == 0
    mesh = plsc.VectorSubcoreMesh(core_axis_name="c", subcore_axis_name="s")

    @functools.partial(
        pl.kernel,
        out_type=jax.ShapeDtypeStruct((_NC, n_pad, d), jnp.float32),
        mesh=mesh,
        scratch_types=[
            pltpu.VMEM((2, _GRP, _CHUNK), jnp.int32),
            pltpu.VMEM((2, _GRP, _CHUNK), jnp.int32),
            pltpu.VMEM((2, _GRP, _CHUNK), jnp.float32),
            pltpu.VMEM((_CHUNK, 128), jnp.float32),
            pltpu.VMEM((_CHUNK, 128), jnp.float32),
            pltpu.VMEM_SHARED((n_pad, 128), jnp.float32),
            pltpu.SemaphoreType.DMA,
            pltpu.SemaphoreType.DMA,
            pltpu.SemaphoreType.DMA,
            pltpu.SemaphoreType.DMA,
        ],
    )
    def k(x_hbm, src_hbm, dst_hbm, ew_hbm, out_hbm,
          src_v, dst_v, ew_v, rows0, rows1, accum,
          semi0, semi1, semg0, semg1):
        c = lax.axis_index("c")
        s = lax.axis_index("s")
        t = c * _NS + s

        isem = (semi0, semi1)
        gbuf = (rows0, rows1)
        gsem = (semg0, semg1)

        def issue_idx(g, slot):
            pltpu.async_copy(src_hbm.at[t, g], src_v.at[slot], isem[slot])
            pltpu.async_copy(dst_hbm.at[t, g], dst_v.at[slot], isem[slot])
            pltpu.async_copy(ew_hbm.at[t, g], ew_v.at[slot], isem[slot])

        def wait_idx(slot):
            pltpu.make_async_copy(src_hbm.at[t, 0], src_v.at[slot], isem[slot]).wait()
            pltpu.make_async_copy(dst_hbm.at[t, 0], dst_v.at[slot], isem[slot]).wait()
            pltpu.make_async_copy(ew_hbm.at[t, 0], ew_v.at[slot], isem[slot]).wait()

        def issue_gather(slot, row, p):
            pltpu.async_copy(x_hbm.at[src_v.at[slot, row]], gbuf[p], gsem[p])

        def wait_gather(slot, row, p):
            pltpu.make_async_copy(
                x_hbm.at[src_v.at[slot, row]], gbuf[p], gsem[p]).wait()

        @pl.loop(0, _CHUNK)
        def _(r):
            for g in range(d // _LANES):
                rows0[r, pl.ds(g * _LANES, _LANES)] = jnp.zeros((_LANES,), jnp.float32)

        rbase = s * rows_per_tile

        @pl.loop(0, rows_per_tile // _CHUNK)
        def _(i):
            pltpu.sync_copy(rows0, accum.at[pl.ds(rbase + i * _CHUNK, _CHUNK)])

        rem = rows_per_tile % _CHUNK
        if rem:
            pltpu.sync_copy(
                rows0.at[pl.ds(0, rem)],
                accum.at[pl.ds(rbase + (rows_per_tile // _CHUNK) * _CHUNK, rem)])

        def scale(slot, jj, buf):
            @pl.loop(0, _CHUNK // _LANES)
            def _(g):
                raw = ew_v[slot, jj, pl.ds(g * _LANES, _LANES)]
                ew16 = 1.0 / (1.0 + jnp.exp(-raw))
                for le in range(_LANES):
                    e = g * _LANES + le
                    w = lax.gather(
                        ew16, jnp.full((_LANES, 1), le, jnp.int32),
                        lax.GatherDimensionNumbers(
                            offset_dims=(), collapsed_slice_dims=(0,),
                            start_index_map=(0,)),
                        (1,), mode=lax.GatherScatterMode.PROMISE_IN_BOUNDS)
                    for kk in range(d // _LANES):
                        sl = pl.ds(kk * _LANES, _LANES)
                        buf[e, sl] = buf[e, sl] * w

        issue_idx(0, 0)
        wait_idx(0)
        issue_gather(0, 0, 0)
        issue_gather(0, 1, 1)

        plsc.subcore_barrier()

        def group_body(g, slot):
            oslot = 1 - slot
            gn = jnp.minimum(g + 1, n_grp - 1)
            issue_idx(gn, oslot)
            for jj in range(_GRP):
                p = jj % 2
                wait_gather(slot, jj, p)
                scale(slot, jj, gbuf[p])
                pltpu.sync_copy(gbuf[p], accum.at[dst_v.at[slot, jj]], add=True)
                if jj == _GRP - 3:
                    wait_idx(oslot)
                if jj < _GRP - 2:
                    issue_gather(slot, jj + 2, p)
                else:
                    issue_gather(oslot, jj - (_GRP - 2), p)

        @pl.loop(0, n_grp, step=2)
        def _(g):
            group_body(g, 0)
            group_body(g + 1, 1)

        wait_gather(0, 0, 0)
        wait_gather(0, 1, 1)

        plsc.subcore_barrier()

        pltpu.sync_copy(accum.at[pl.ds(rbase, rows_per_tile)],
                        out_hbm.at[c].at[pl.ds(rbase, rows_per_tile)])

    return k(x, src4d, dst4d, ew4d)


def _tc_layer(aggp, inp, w_rel, b_rel, w_root, relu):
    n, d = inp.shape
    o = w_rel.shape[1]

    def body(aggp_ref, x_ref, wrel_ref, b_ref, wroot_ref, o_ref):
        agg = aggp_ref[0] + aggp_ref[1]
        r = (jnp.dot(agg, wrel_ref[...], preferred_element_type=jnp.float32,
                     precision=lax.Precision.HIGHEST)
             + jnp.dot(x_ref[...], wroot_ref[...],
                       preferred_element_type=jnp.float32,
                       precision=lax.Precision.HIGHEST)
             + b_ref[...])
        o_ref[...] = jnp.maximum(r, 0.0) if relu else r

    return pl.pallas_call(
        body,
        out_shape=jax.ShapeDtypeStruct((n, o), jnp.float32),
    )(aggp, inp, w_rel, b_rel.reshape(1, o), w_root)


def kernel(x, edge_index, edge_weight, W1_rel, b1_rel, W1_root,
           W2_rel, b2_rel, W2_root):
    n, d = x.shape
    e = edge_weight.shape[0]
    cpt = -(-e // (_NW * _CHUNK))
    cpt = -(-cpt // (2 * _GRP)) * (2 * _GRP)
    e_pad = _NW * cpt * _CHUNK
    n_grp = cpt // _GRP

    shp = (_NW, n_grp, _GRP, _CHUNK)
    src = jnp.pad(edge_index[0], (0, e_pad - e)).reshape(shp)
    dst = jnp.pad(edge_index[1], (0, e_pad - e)).reshape(shp)
    ewp = jnp.pad(edge_weight, (0, e_pad - e),
                  constant_values=-1e30).reshape(shp)

    n_pad = -(-n // (_NS * 8)) * (_NS * 8)

    p1 = _sc_segment_sum(x, src, dst, ewp, n_pad, cpt)[:, :n, :]
    h = _tc_layer(p1, x, W1_rel, b1_rel, W1_root, relu=True)
    p2 = _sc_segment_sum(h, src, dst, ewp, n_pad, cpt)[:, :n, :]
    out = _tc_layer(p2, h, W2_rel, b2_rel, W2_root, relu=False)
    return out
